# baseline (speedup 1.0000x reference)
import functools

import jax
import jax.numpy as jnp
from jax import lax
from jax.experimental import pallas as pl
from jax.experimental.pallas import tpu as pltpu
from jax.experimental.pallas import tpu_sc as plsc

N_NODES = 10000
N_TILES = 16
ROWS_PER_TILE = 640
N_PAD = N_TILES * ROWS_PER_TILE
T_WIN = 24
FEAT = 128
HID = 64
GH = 32
NG = T_WIN // 2
GSTEPS = NG // 2
ROWW = 2 * HID
NPASS = 4
NCHH = 40
NCH = NPASS * NCHH
E_PAD = N_TILES * NCH * 128
BN = 400
NB = N_NODES // BN


def _sc_mesh():
    return plsc.VectorSubcoreMesh(core_axis_name="c", subcore_axis_name="s")


def _deg_body(dst_hbm, ones_hbm, zero_hbm, out_hbm,
              dst_v, ones_v, zero_v, deg_sh, sem):
    c = lax.axis_index("c")
    s = lax.axis_index("s")
    pltpu.sync_copy(ones_hbm, ones_v)
    pltpu.sync_copy(zero_hbm, zero_v)
    row0 = s * ROWS_PER_TILE
    for k in range(ROWS_PER_TILE // 128):
        pltpu.sync_copy(zero_v, deg_sh.at[pl.ds(row0 + k * 128, 128)])
    plsc.subcore_barrier()

    def chunk(j, carry):
        pltpu.async_copy(ones_v, deg_sh.at[dst_v.at[j]], sem, add=True).wait()
        return carry

    for p in range(NPASS):
        pltpu.sync_copy(dst_hbm.at[s, p], dst_v)
        lax.fori_loop(0, NCHH, chunk, 0)
    plsc.subcore_barrier()

    for k in range(ROWS_PER_TILE // 128):
        r = row0 + k * 128
        pltpu.sync_copy(deg_sh.at[pl.ds(r, 128)], out_hbm.at[c, pl.ds(r, 128)])


def _deg_call(dst_idx, ones128, zeros128):
    return pl.kernel(
        _deg_body,
        out_type=jax.ShapeDtypeStruct((2, N_PAD, 128), jnp.float32),
        mesh=_sc_mesh(),
        scratch_types=[
            pltpu.VMEM((NCHH, 128), jnp.int32),
            pltpu.VMEM((128, 128), jnp.float32),
            pltpu.VMEM((128, 128), jnp.float32),
            pltpu.VMEM_SHARED((N_PAD, 128), jnp.float32),
            pltpu.SemaphoreType.DMA,
        ],
    )(dst_idx, ones128, zeros128)


def _agg_body(z_hbm, src_hbm, dst_hbm, zero_hbm, out_hbm,
              src_v, dst_v, rows_v, agg_sh, gsem, ssem):
    c = lax.axis_index("c")
    s = lax.axis_index("s")
    row0 = s * ROWS_PER_TILE
    for gstep in range(GSTEPS):
        g = gstep * 2 + c
        for k in range(ROWS_PER_TILE // 128):
            pltpu.sync_copy(zero_hbm, agg_sh.at[pl.ds(row0 + k * 128, 128)])
        plsc.subcore_barrier()

        def wait_g():
            pltpu.make_async_copy(zero_hbm, rows_v.at[0], gsem).wait()

        def wait_s():
            pltpu.make_async_copy(zero_hbm, rows_v.at[0], ssem).wait()

        def start_gather(j, b):
            pltpu.async_copy(z_hbm.at[src_v.at[j]], rows_v.at[b], gsem)

        def start_scatter(j, b):
            pltpu.async_copy(rows_v.at[b], agg_sh.at[dst_v.at[j]], ssem,
                             add=True)

        def steady(j, carry):
            b = jnp.bitwise_and(j, 1)
            wait_g()
            wait_s()
            start_gather(j + 1, 1 - b)
            start_scatter(j, b)
            return carry

        for p in range(NPASS):
            pltpu.sync_copy(src_hbm.at[gstep, c, s, p], src_v)
            pltpu.sync_copy(dst_hbm.at[s, p], dst_v)
            start_gather(0, 0)
            wait_g()
            start_gather(1, 1)
            start_scatter(0, 0)
            lax.fori_loop(1, NCHH - 1, steady, 0)
            wait_g()
            wait_s()
            start_scatter(NCHH - 1, (NCHH - 1) & 1)
            wait_s()
        plsc.subcore_barrier()
        for k in range(ROWS_PER_TILE // 128):
            r = row0 + k * 128
            pltpu.sync_copy(agg_sh.at[pl.ds(r, 128)], out_hbm.at[g, pl.ds(r, 128)])


@functools.lru_cache(maxsize=None)
def _agg_kernel():
    return pl.kernel(
        _agg_body,
        out_type=jax.ShapeDtypeStruct((NG, N_PAD, ROWW), jnp.float32),
        mesh=_sc_mesh(),
        scratch_types=[
            pltpu.VMEM((NCHH, 128), jnp.int32),
            pltpu.VMEM((NCHH, 128), jnp.int32),
            pltpu.VMEM((2, 128, ROWW), jnp.float32),
            pltpu.VMEM_SHARED((N_PAD, ROWW), jnp.float32),
            pltpu.SemaphoreType.DMA,
            pltpu.SemaphoreType.DMA,
        ],
    )


def _agg_call(z_flat, src_idx, dst_idx, zeros128):
    return _agg_kernel()(z_flat, src_idx, dst_idx, zeros128)


def _k1_body(x_ref, wc_ref, bc_ref, deg_ref, out_ref):
    dinv = lax.rsqrt(deg_ref[:, 0:1] + 1.0)
    xb = x_ref[0]
    for t in range(T_WIN):
        z = jnp.dot(xb[:, t, :], wc_ref[...],
                    preferred_element_type=jnp.float32) + bc_ref[0]
        z = z * dinv
        g, h = divmod(t, 2)
        out_ref[g, :, h * HID:(h + 1) * HID] = z


def _k1_call(x, Wc, bc, deg):
    return pl.pallas_call(
        _k1_body,
        grid=(NB,),
        in_specs=[
            pl.BlockSpec((1, BN, T_WIN, FEAT), lambda nb: (0, nb, 0, 0)),
            pl.BlockSpec((FEAT, HID), lambda nb: (0, 0)),
            pl.BlockSpec((1, HID), lambda nb: (0, 0)),
            pl.BlockSpec((BN, 128), lambda nb: (nb, 0)),
        ],
        out_specs=pl.BlockSpec((NG, BN, ROWW), lambda nb: (0, nb, 0)),
        out_shape=jax.ShapeDtypeStruct((NG, N_PAD, ROWW), jnp.float32),
    )(x, Wc, bc, deg)


def _k3_body(agg_ref, z_ref, deg_ref, b_ref, wd_ref, out_ref):
    dinv = lax.rsqrt(deg_ref[:, 0:1] + 1.0)
    h = jnp.maximum((agg_ref[0] + z_ref[0]) * dinv + b_ref[0], 0.0)
    out_ref[0] = jnp.dot(h, wd_ref[...],
                         preferred_element_type=jnp.float32) * dinv


def _k3_call(agg, z, deg, bias, WdT):
    return pl.pallas_call(
        _k3_body,
        grid=(NG, NB),
        in_specs=[
            pl.BlockSpec((1, BN, ROWW), lambda g, nb: (g, nb, 0)),
            pl.BlockSpec((1, BN, ROWW), lambda g, nb: (g, nb, 0)),
            pl.BlockSpec((BN, 128), lambda g, nb: (nb, 0)),
            pl.BlockSpec((1, ROWW), lambda g, nb: (0, 0)),
            pl.BlockSpec((ROWW, ROWW), lambda g, nb: (0, 0)),
        ],
        out_specs=pl.BlockSpec((1, BN, ROWW), lambda g, nb: (g, nb, 0)),
        out_shape=jax.ShapeDtypeStruct((NG, N_PAD, ROWW), jnp.float32),
    )(agg, z, deg, bias, WdT)


def _k4_body(agg_ref, z_ref, deg_ref, b2_ref,
             wih0_ref, whh0_ref, bih0_ref, bhh0_ref,
             wih1_ref, whh1_ref, bih1_ref, bhh1_ref,
             w1d_ref, b1d_ref, w2d_ref, b2d_ref, out_ref):
    f32 = jnp.float32
    dinv = lax.rsqrt(deg_ref[:, 0:1] + 1.0)
    h3 = [jnp.maximum((agg_ref[g] + z_ref[g]) * dinv + b2_ref[0], 0.0)
          for g in range(NG)]
    h1 = jnp.zeros((BN, GH), f32)
    h2 = jnp.zeros((BN, GH), f32)
    for t in range(T_WIN):
        g, hf = divmod(t, 2)
        xt = h3[g][:, hf * HID:(hf + 1) * HID]
        gi = jnp.dot(xt, wih0_ref[...], preferred_element_type=f32) + bih0_ref[0]
        gh = jnp.dot(h1, whh0_ref[...], preferred_element_type=f32) + bhh0_ref[0]
        r = jax.nn.sigmoid(gi[:, 0:GH] + gh[:, 0:GH])
        zg = jax.nn.sigmoid(gi[:, GH:2 * GH] + gh[:, GH:2 * GH])
        n = jnp.tanh(gi[:, 2 * GH:3 * GH] + r * gh[:, 2 * GH:3 * GH])
        h1 = (1.0 - zg) * n + zg * h1
        gi2 = jnp.dot(h1, wih1_ref[...], preferred_element_type=f32) + bih1_ref[0]
        gh2 = jnp.dot(h2, whh1_ref[...], preferred_element_type=f32) + bhh1_ref[0]
        r2 = jax.nn.sigmoid(gi2[:, 0:GH] + gh2[:, 0:GH])
        zg2 = jax.nn.sigmoid(gi2[:, GH:2 * GH] + gh2[:, GH:2 * GH])
        n2 = jnp.tanh(gi2[:, 2 * GH:3 * GH] + r2 * gh2[:, 2 * GH:3 * GH])
        h2 = (1.0 - zg2) * n2 + zg2 * h2
        hd = jnp.maximum(jnp.dot(h2, w1d_ref[...], preferred_element_type=f32)
                         + b1d_ref[0], 0.0)
        ot = jnp.dot(hd, w2d_ref[...], preferred_element_type=f32) + b2d_ref[0]
        out_ref[0, :, t, :] = ot


def _k4_call(agg, z, deg, bias2, gw):
    full = lambda *shape: None
    return pl.pallas_call(
        _k4_body,
        grid=(NB,),
        in_specs=[
            pl.BlockSpec((NG, BN, ROWW), lambda nb: (0, nb, 0)),
            pl.BlockSpec((NG, BN, ROWW), lambda nb: (0, nb, 0)),
            pl.BlockSpec((BN, 128), lambda nb: (nb, 0)),
            pl.BlockSpec((1, ROWW), lambda nb: (0, 0)),
            pl.BlockSpec((HID, 3 * GH), lambda nb: (0, 0)),
            pl.BlockSpec((GH, 3 * GH), lambda nb: (0, 0)),
            pl.BlockSpec((1, 3 * GH), lambda nb: (0, 0)),
            pl.BlockSpec((1, 3 * GH), lambda nb: (0, 0)),
            pl.BlockSpec((GH, 3 * GH), lambda nb: (0, 0)),
            pl.BlockSpec((GH, 3 * GH), lambda nb: (0, 0)),
            pl.BlockSpec((1, 3 * GH), lambda nb: (0, 0)),
            pl.BlockSpec((1, 3 * GH), lambda nb: (0, 0)),
            pl.BlockSpec((GH, 96), lambda nb: (0, 0)),
            pl.BlockSpec((1, 96), lambda nb: (0, 0)),
            pl.BlockSpec((96, 12), lambda nb: (0, 0)),
            pl.BlockSpec((1, 12), lambda nb: (0, 0)),
        ],
        out_specs=pl.BlockSpec((1, BN, T_WIN, 12), lambda nb: (0, nb, 0, 0)),
        out_shape=jax.ShapeDtypeStruct((1, N_NODES, T_WIN, 12), jnp.float32),
    )(agg, z, deg, bias2, *gw)


def kernel(x, edge_index, proj_W, proj_b, gcn_W, gcn_b,
           gW_ih0, gW_hh0, gb_ih0, gb_hh0, gW_ih1, gW_hh1, gb_ih1, gb_hh1,
           dec_W1, dec_b1, dec_W2, dec_b2):
    f32 = jnp.float32
    x = x.astype(f32)
    src = edge_index[0].astype(jnp.int32)
    dst = edge_index[1].astype(jnp.int32)
    E = src.shape[0]
    pad = E_PAD - E
    src_p = jnp.concatenate([src, jnp.zeros((pad,), jnp.int32)])
    dst_p = jnp.concatenate([dst, jnp.full((pad,), N_NODES, jnp.int32)])
    dst_idx = dst_p.reshape(N_TILES, NPASS, NCHH, 128)
    goff = (jnp.arange(NG, dtype=jnp.int32) * N_PAD).reshape(GSTEPS, 2, 1, 1, 1, 1)
    src_idx = src_p.reshape(1, 1, N_TILES, NPASS, NCHH, 128) + goff
    zeros128 = jnp.zeros((128, 128), f32)
    ones128 = jnp.ones((128, 128), f32)

    deg = _deg_call(dst_idx, ones128, zeros128)[0]

    Wc = proj_W.T @ gcn_W[0].T
    bc = (proj_b @ gcn_W[0].T).reshape(1, HID)
    Z = _k1_call(x, Wc, bc, deg)
    for l in (0, 1):
        AGG = _agg_call(Z.reshape(NG * N_PAD, ROWW), src_idx, dst_idx, zeros128)
        WdT = jnp.kron(jnp.eye(2, dtype=f32), gcn_W[l + 1].T)
        bias = jnp.tile(gcn_b[l], 2).reshape(1, ROWW)
        Z = _k3_call(AGG, Z, deg, bias, WdT)
    AGG = _agg_call(Z.reshape(NG * N_PAD, ROWW), src_idx, dst_idx, zeros128)
    bias2 = jnp.tile(gcn_b[2], 2).reshape(1, ROWW)

    import jax.scipy.linalg as jsl
    segs = [(0, 1), (1, 2), (2, 3), (3, 6), (6, 9), (9, 12)]
    gw = [
        gW_ih0.T, gW_hh0.T, gb_ih0.reshape(1, -1), gb_hh0.reshape(1, -1),
        gW_ih1.T, gW_hh1.T, gb_ih1.reshape(1, -1), gb_hh1.reshape(1, -1),
        dec_W1.reshape(6 * (GH // 2), GH).T, dec_b1.reshape(1, -1),
        jsl.block_diag(*[dec_W2[s0:s1].T for (s0, s1) in segs]),
        dec_b2.reshape(1, 12),
    ]
    return _k4_call(AGG, Z, deg, bias2, gw)


# speedup vs baseline: 1.0056x; 1.0056x over previous
import functools

import jax
import jax.numpy as jnp
from jax import lax
from jax.experimental import pallas as pl
from jax.experimental.pallas import tpu as pltpu
from jax.experimental.pallas import tpu_sc as plsc

N_NODES = 10000
N_TILES = 16
ROWS_PER_TILE = 640
N_PAD = N_TILES * ROWS_PER_TILE
T_WIN = 24
FEAT = 128
HID = 64
GH = 32
NG = T_WIN // 2
GSTEPS = NG // 2
ROWW = 2 * HID
NPASS = 4
NCHH = 40
NCH = NPASS * NCHH
E_PAD = N_TILES * NCH * 128
BN = 400
NB = N_NODES // BN


def _sc_mesh():
    return plsc.VectorSubcoreMesh(core_axis_name="c", subcore_axis_name="s")


def _deg_body(dst_hbm, ones_hbm, zero_hbm, out_hbm,
              dst_v, ones_v, zero_v, deg_sh, sem):
    c = lax.axis_index("c")
    s = lax.axis_index("s")
    pltpu.sync_copy(ones_hbm, ones_v)
    pltpu.sync_copy(zero_hbm, zero_v)
    row0 = s * ROWS_PER_TILE
    for k in range(ROWS_PER_TILE // 128):
        pltpu.sync_copy(zero_v, deg_sh.at[pl.ds(row0 + k * 128, 128)])
    plsc.subcore_barrier()

    def chunk(j, carry):
        pltpu.async_copy(ones_v, deg_sh.at[dst_v.at[j]], sem, add=True).wait()
        return carry

    for p in range(NPASS):
        pltpu.sync_copy(dst_hbm.at[s, p], dst_v)
        lax.fori_loop(0, NCHH, chunk, 0)
    plsc.subcore_barrier()

    for k in range(ROWS_PER_TILE // 128):
        r = row0 + k * 128
        pltpu.sync_copy(deg_sh.at[pl.ds(r, 128)], out_hbm.at[c, pl.ds(r, 128)])


def _deg_call(dst_idx, ones128, zeros128):
    return pl.kernel(
        _deg_body,
        out_type=jax.ShapeDtypeStruct((2, N_PAD, 128), jnp.float32),
        mesh=_sc_mesh(),
        scratch_types=[
            pltpu.VMEM((NCHH, 128), jnp.int32),
            pltpu.VMEM((128, 128), jnp.float32),
            pltpu.VMEM((128, 128), jnp.float32),
            pltpu.VMEM_SHARED((N_PAD, 128), jnp.float32),
            pltpu.SemaphoreType.DMA,
        ],
    )(dst_idx, ones128, zeros128)


def _agg_body(z_hbm, src_hbm, dst_hbm, zero_hbm, out_hbm,
              src_v, dst_v, rows_v, agg_sh, gsem, ssem):
    c = lax.axis_index("c")
    s = lax.axis_index("s")
    row0 = s * ROWS_PER_TILE
    for gstep in range(GSTEPS):
        g = gstep * 2 + c
        for k in range(ROWS_PER_TILE // 128):
            pltpu.sync_copy(zero_hbm, agg_sh.at[pl.ds(row0 + k * 128, 128)])
        plsc.subcore_barrier()

        def wait_g():
            pltpu.make_async_copy(zero_hbm, rows_v.at[0], gsem).wait()

        def start_gather(j, b):
            pltpu.async_copy(z_hbm.at[src_v.at[j]], rows_v.at[b], gsem)

        def steady(j, carry):
            b = jnp.bitwise_and(j, 1)
            wait_g()
            start_gather(j + 1, 1 - b)
            pltpu.async_copy(rows_v.at[b], agg_sh.at[dst_v.at[j]], ssem,
                             add=True).wait()
            return carry

        for p in range(NPASS):
            pltpu.sync_copy(src_hbm.at[gstep, c, s, p], src_v)
            pltpu.sync_copy(dst_hbm.at[s, p], dst_v)
            start_gather(0, 0)
            lax.fori_loop(0, NCHH - 1, steady, 0)
            wait_g()
            b = (NCHH - 1) & 1
            pltpu.async_copy(rows_v.at[b], agg_sh.at[dst_v.at[NCHH - 1]],
                             ssem, add=True).wait()
        plsc.subcore_barrier()
        for k in range(ROWS_PER_TILE // 128):
            r = row0 + k * 128
            pltpu.sync_copy(agg_sh.at[pl.ds(r, 128)], out_hbm.at[g, pl.ds(r, 128)])


@functools.lru_cache(maxsize=None)
def _agg_kernel():
    return pl.kernel(
        _agg_body,
        out_type=jax.ShapeDtypeStruct((NG, N_PAD, ROWW), jnp.float32),
        mesh=_sc_mesh(),
        scratch_types=[
            pltpu.VMEM((NCHH, 128), jnp.int32),
            pltpu.VMEM((NCHH, 128), jnp.int32),
            pltpu.VMEM((2, 128, ROWW), jnp.float32),
            pltpu.VMEM_SHARED((N_PAD, ROWW), jnp.float32),
            pltpu.SemaphoreType.DMA,
            pltpu.SemaphoreType.DMA,
        ],
    )


def _agg_call(z_flat, src_idx, dst_idx, zeros128):
    return _agg_kernel()(z_flat, src_idx, dst_idx, zeros128)


def _k1_body(x_ref, wc_ref, bc_ref, deg_ref, out_ref):
    dinv = lax.rsqrt(deg_ref[:, 0:1] + 1.0)
    xb = x_ref[0]
    for t in range(T_WIN):
        z = jnp.dot(xb[:, t, :], wc_ref[...],
                    preferred_element_type=jnp.float32) + bc_ref[0]
        z = z * dinv
        g, h = divmod(t, 2)
        out_ref[g, :, h * HID:(h + 1) * HID] = z


def _k1_call(x, Wc, bc, deg):
    return pl.pallas_call(
        _k1_body,
        grid=(NB,),
        in_specs=[
            pl.BlockSpec((1, BN, T_WIN, FEAT), lambda nb: (0, nb, 0, 0)),
            pl.BlockSpec((FEAT, HID), lambda nb: (0, 0)),
            pl.BlockSpec((1, HID), lambda nb: (0, 0)),
            pl.BlockSpec((BN, 128), lambda nb: (nb, 0)),
        ],
        out_specs=pl.BlockSpec((NG, BN, ROWW), lambda nb: (0, nb, 0)),
        out_shape=jax.ShapeDtypeStruct((NG, N_PAD, ROWW), jnp.float32),
    )(x, Wc, bc, deg)


def _k3_body(agg_ref, z_ref, deg_ref, b_ref, wd_ref, out_ref):
    dinv = lax.rsqrt(deg_ref[:, 0:1] + 1.0)
    h = jnp.maximum((agg_ref[0] + z_ref[0]) * dinv + b_ref[0], 0.0)
    out_ref[0] = jnp.dot(h, wd_ref[...],
                         preferred_element_type=jnp.float32) * dinv


def _k3_call(agg, z, deg, bias, WdT):
    return pl.pallas_call(
        _k3_body,
        grid=(NG, NB),
        in_specs=[
            pl.BlockSpec((1, BN, ROWW), lambda g, nb: (g, nb, 0)),
            pl.BlockSpec((1, BN, ROWW), lambda g, nb: (g, nb, 0)),
            pl.BlockSpec((BN, 128), lambda g, nb: (nb, 0)),
            pl.BlockSpec((1, ROWW), lambda g, nb: (0, 0)),
            pl.BlockSpec((ROWW, ROWW), lambda g, nb: (0, 0)),
        ],
        out_specs=pl.BlockSpec((1, BN, ROWW), lambda g, nb: (g, nb, 0)),
        out_shape=jax.ShapeDtypeStruct((NG, N_PAD, ROWW), jnp.float32),
    )(agg, z, deg, bias, WdT)


def _k4_body(agg_ref, z_ref, deg_ref, b2_ref,
             wih0_ref, whh0_ref, bih0_ref, bhh0_ref,
             wih1_ref, whh1_ref, bih1_ref, bhh1_ref,
             w1d_ref, b1d_ref, w2d_ref, b2d_ref, out_ref):
    f32 = jnp.float32
    dinv = lax.rsqrt(deg_ref[:, 0:1] + 1.0)
    h3 = [jnp.maximum((agg_ref[g] + z_ref[g]) * dinv + b2_ref[0], 0.0)
          for g in range(NG)]
    h1 = jnp.zeros((BN, GH), f32)
    h2 = jnp.zeros((BN, GH), f32)
    for t in range(T_WIN):
        g, hf = divmod(t, 2)
        xt = h3[g][:, hf * HID:(hf + 1) * HID]
        gi = jnp.dot(xt, wih0_ref[...], preferred_element_type=f32) + bih0_ref[0]
        gh = jnp.dot(h1, whh0_ref[...], preferred_element_type=f32) + bhh0_ref[0]
        r = jax.nn.sigmoid(gi[:, 0:GH] + gh[:, 0:GH])
        zg = jax.nn.sigmoid(gi[:, GH:2 * GH] + gh[:, GH:2 * GH])
        n = jnp.tanh(gi[:, 2 * GH:3 * GH] + r * gh[:, 2 * GH:3 * GH])
        h1 = (1.0 - zg) * n + zg * h1
        gi2 = jnp.dot(h1, wih1_ref[...], preferred_element_type=f32) + bih1_ref[0]
        gh2 = jnp.dot(h2, whh1_ref[...], preferred_element_type=f32) + bhh1_ref[0]
        r2 = jax.nn.sigmoid(gi2[:, 0:GH] + gh2[:, 0:GH])
        zg2 = jax.nn.sigmoid(gi2[:, GH:2 * GH] + gh2[:, GH:2 * GH])
        n2 = jnp.tanh(gi2[:, 2 * GH:3 * GH] + r2 * gh2[:, 2 * GH:3 * GH])
        h2 = (1.0 - zg2) * n2 + zg2 * h2
        hd = jnp.maximum(jnp.dot(h2, w1d_ref[...], preferred_element_type=f32)
                         + b1d_ref[0], 0.0)
        ot = jnp.dot(hd, w2d_ref[...], preferred_element_type=f32) + b2d_ref[0]
        out_ref[0, :, t, :] = ot


def _k4_call(agg, z, deg, bias2, gw):
    full = lambda *shape: None
    return pl.pallas_call(
        _k4_body,
        grid=(NB,),
        in_specs=[
            pl.BlockSpec((NG, BN, ROWW), lambda nb: (0, nb, 0)),
            pl.BlockSpec((NG, BN, ROWW), lambda nb: (0, nb, 0)),
            pl.BlockSpec((BN, 128), lambda nb: (nb, 0)),
            pl.BlockSpec((1, ROWW), lambda nb: (0, 0)),
            pl.BlockSpec((HID, 3 * GH), lambda nb: (0, 0)),
            pl.BlockSpec((GH, 3 * GH), lambda nb: (0, 0)),
            pl.BlockSpec((1, 3 * GH), lambda nb: (0, 0)),
            pl.BlockSpec((1, 3 * GH), lambda nb: (0, 0)),
            pl.BlockSpec((GH, 3 * GH), lambda nb: (0, 0)),
            pl.BlockSpec((GH, 3 * GH), lambda nb: (0, 0)),
            pl.BlockSpec((1, 3 * GH), lambda nb: (0, 0)),
            pl.BlockSpec((1, 3 * GH), lambda nb: (0, 0)),
            pl.BlockSpec((GH, 96), lambda nb: (0, 0)),
            pl.BlockSpec((1, 96), lambda nb: (0, 0)),
            pl.BlockSpec((96, 12), lambda nb: (0, 0)),
            pl.BlockSpec((1, 12), lambda nb: (0, 0)),
        ],
        out_specs=pl.BlockSpec((1, BN, T_WIN, 12), lambda nb: (0, nb, 0, 0)),
        out_shape=jax.ShapeDtypeStruct((1, N_NODES, T_WIN, 12), jnp.float32),
    )(agg, z, deg, bias2, *gw)


def kernel(x, edge_index, proj_W, proj_b, gcn_W, gcn_b,
           gW_ih0, gW_hh0, gb_ih0, gb_hh0, gW_ih1, gW_hh1, gb_ih1, gb_hh1,
           dec_W1, dec_b1, dec_W2, dec_b2):
    f32 = jnp.float32
    x = x.astype(f32)
    src = edge_index[0].astype(jnp.int32)
    dst = edge_index[1].astype(jnp.int32)
    E = src.shape[0]
    pad = E_PAD - E
    src_p = jnp.concatenate([src, jnp.zeros((pad,), jnp.int32)])
    dst_p = jnp.concatenate([dst, jnp.full((pad,), N_NODES, jnp.int32)])
    dst_idx = dst_p.reshape(N_TILES, NPASS, NCHH, 128)
    goff = (jnp.arange(NG, dtype=jnp.int32) * N_PAD).reshape(GSTEPS, 2, 1, 1, 1, 1)
    src_idx = src_p.reshape(1, 1, N_TILES, NPASS, NCHH, 128) + goff
    zeros128 = jnp.zeros((128, 128), f32)
    ones128 = jnp.ones((128, 128), f32)

    deg = _deg_call(dst_idx, ones128, zeros128)[0]

    Wc = proj_W.T @ gcn_W[0].T
    bc = (proj_b @ gcn_W[0].T).reshape(1, HID)
    Z = _k1_call(x, Wc, bc, deg)
    for l in (0, 1):
        AGG = _agg_call(Z.reshape(NG * N_PAD, ROWW), src_idx, dst_idx, zeros128)
        WdT = jnp.kron(jnp.eye(2, dtype=f32), gcn_W[l + 1].T)
        bias = jnp.tile(gcn_b[l], 2).reshape(1, ROWW)
        Z = _k3_call(AGG, Z, deg, bias, WdT)
    AGG = _agg_call(Z.reshape(NG * N_PAD, ROWW), src_idx, dst_idx, zeros128)
    bias2 = jnp.tile(gcn_b[2], 2).reshape(1, ROWW)

    import jax.scipy.linalg as jsl
    segs = [(0, 1), (1, 2), (2, 3), (3, 6), (6, 9), (9, 12)]
    gw = [
        gW_ih0.T, gW_hh0.T, gb_ih0.reshape(1, -1), gb_hh0.reshape(1, -1),
        gW_ih1.T, gW_hh1.T, gb_ih1.reshape(1, -1), gb_hh1.reshape(1, -1),
        dec_W1.reshape(6 * (GH // 2), GH).T, dec_b1.reshape(1, -1),
        jsl.block_diag(*[dec_W2[s0:s1].T for (s0, s1) in segs]),
        dec_b2.reshape(1, 12),
    ]
    return _k4_call(AGG, Z, deg, bias2, gw)


# speedup vs baseline: 1.2098x; 1.2030x over previous
import functools

import jax
import jax.numpy as jnp
from jax import lax
from jax.experimental import pallas as pl
from jax.experimental.pallas import tpu as pltpu
from jax.experimental.pallas import tpu_sc as plsc

N_NODES = 10000
N_TILES = 16
ROWS_PER_TILE = 640
N_PAD = N_TILES * ROWS_PER_TILE
T_WIN = 24
FEAT = 128
HID = 64
GH = 32
NG = T_WIN // 2
GSTEPS = NG // 2
ROWW = 2 * HID
NPASS = 2
NCHH = 79
NCH = NPASS * NCHH
E_PAD = N_TILES * NCH * 128
BN = 400
NB = N_NODES // BN


def _sc_mesh():
    return plsc.VectorSubcoreMesh(core_axis_name="c", subcore_axis_name="s")


def _deg_body(dst_hbm, ones_hbm, zero_hbm, out_hbm,
              dst_v, ones_v, zero_v, deg_sh, sem):
    c = lax.axis_index("c")
    s = lax.axis_index("s")
    pltpu.sync_copy(ones_hbm, ones_v)
    pltpu.sync_copy(zero_hbm, zero_v)
    row0 = s * ROWS_PER_TILE
    for k in range(ROWS_PER_TILE // 128):
        pltpu.sync_copy(zero_v, deg_sh.at[pl.ds(row0 + k * 128, 128)])
    plsc.subcore_barrier()

    def chunk(j, carry):
        pltpu.async_copy(ones_v, deg_sh.at[dst_v.at[j]], sem, add=True).wait()
        return carry

    for p in range(NPASS):
        pltpu.sync_copy(dst_hbm.at[s, p], dst_v)
        lax.fori_loop(0, NCHH, chunk, 0)
    plsc.subcore_barrier()

    for k in range(ROWS_PER_TILE // 128):
        r = row0 + k * 128
        pltpu.sync_copy(deg_sh.at[pl.ds(r, 128)], out_hbm.at[c, pl.ds(r, 128)])


def _deg_call(dst_idx, ones128, zeros128):
    return pl.kernel(
        _deg_body,
        out_type=jax.ShapeDtypeStruct((2, N_PAD, 128), jnp.float32),
        mesh=_sc_mesh(),
        scratch_types=[
            pltpu.VMEM((NCHH, 128), jnp.int32),
            pltpu.VMEM((128, 128), jnp.float32),
            pltpu.VMEM((128, 128), jnp.float32),
            pltpu.VMEM_SHARED((N_PAD, 128), jnp.float32),
            pltpu.SemaphoreType.DMA,
        ],
    )(dst_idx, ones128, zeros128)


def _agg_body(z_hbm, src_hbm, dst_hbm, zero_hbm, out_hbm,
              src_v, dst_v, rows_v, agg_sh, gsem, ssem):
    c = lax.axis_index("c")
    s = lax.axis_index("s")
    row0 = s * ROWS_PER_TILE
    for gstep in range(GSTEPS):
        g = gstep * 2 + c
        for k in range(ROWS_PER_TILE // 128):
            pltpu.sync_copy(zero_hbm, agg_sh.at[pl.ds(row0 + k * 128, 128)])
        plsc.subcore_barrier()

        def chunk(j, carry):
            pltpu.async_copy(z_hbm.at[src_v.at[j]], rows_v, gsem).wait()
            pltpu.async_copy(rows_v, agg_sh.at[dst_v.at[j]], ssem,
                             add=True).wait()
            return carry

        for p in range(NPASS):
            pltpu.sync_copy(src_hbm.at[gstep, c, s, p], src_v)
            pltpu.sync_copy(dst_hbm.at[s, p], dst_v)
            lax.fori_loop(0, NCHH, chunk, 0)
        plsc.subcore_barrier()
        for k in range(ROWS_PER_TILE // 128):
            r = row0 + k * 128
            pltpu.sync_copy(agg_sh.at[pl.ds(r, 128)], out_hbm.at[g, pl.ds(r, 128)])


@functools.lru_cache(maxsize=None)
def _agg_kernel():
    return pl.kernel(
        _agg_body,
        out_type=jax.ShapeDtypeStruct((NG, N_PAD, ROWW), jnp.float32),
        mesh=_sc_mesh(),
        scratch_types=[
            pltpu.VMEM((NCHH, 128), jnp.int32),
            pltpu.VMEM((NCHH, 128), jnp.int32),
            pltpu.VMEM((128, ROWW), jnp.float32),
            pltpu.VMEM_SHARED((N_PAD, ROWW), jnp.float32),
            pltpu.SemaphoreType.DMA,
            pltpu.SemaphoreType.DMA,
        ],
    )


def _agg_call(z_flat, src_idx, dst_idx, zeros128):
    return _agg_kernel()(z_flat, src_idx, dst_idx, zeros128)


def _k1_body(x_ref, wc_ref, bc_ref, deg_ref, out_ref):
    dinv = lax.rsqrt(deg_ref[:, 0:1] + 1.0)
    xb = x_ref[0]
    for t in range(T_WIN):
        z = jnp.dot(xb[:, t, :], wc_ref[...],
                    preferred_element_type=jnp.float32) + bc_ref[0]
        z = z * dinv
        g, h = divmod(t, 2)
        out_ref[g, :, h * HID:(h + 1) * HID] = z


def _k1_call(x, Wc, bc, deg):
    return pl.pallas_call(
        _k1_body,
        grid=(NB,),
        in_specs=[
            pl.BlockSpec((1, BN, T_WIN, FEAT), lambda nb: (0, nb, 0, 0)),
            pl.BlockSpec((FEAT, HID), lambda nb: (0, 0)),
            pl.BlockSpec((1, HID), lambda nb: (0, 0)),
            pl.BlockSpec((BN, 128), lambda nb: (nb, 0)),
        ],
        out_specs=pl.BlockSpec((NG, BN, ROWW), lambda nb: (0, nb, 0)),
        out_shape=jax.ShapeDtypeStruct((NG, N_PAD, ROWW), jnp.float32),
    )(x, Wc, bc, deg)


def _k3_body(agg_ref, z_ref, deg_ref, b_ref, wd_ref, out_ref):
    dinv = lax.rsqrt(deg_ref[:, 0:1] + 1.0)
    h = jnp.maximum((agg_ref[0] + z_ref[0]) * dinv + b_ref[0], 0.0)
    out_ref[0] = jnp.dot(h, wd_ref[...],
                         preferred_element_type=jnp.float32) * dinv


def _k3_call(agg, z, deg, bias, WdT):
    return pl.pallas_call(
        _k3_body,
        grid=(NG, NB),
        in_specs=[
            pl.BlockSpec((1, BN, ROWW), lambda g, nb: (g, nb, 0)),
            pl.BlockSpec((1, BN, ROWW), lambda g, nb: (g, nb, 0)),
            pl.BlockSpec((BN, 128), lambda g, nb: (nb, 0)),
            pl.BlockSpec((1, ROWW), lambda g, nb: (0, 0)),
            pl.BlockSpec((ROWW, ROWW), lambda g, nb: (0, 0)),
        ],
        out_specs=pl.BlockSpec((1, BN, ROWW), lambda g, nb: (g, nb, 0)),
        out_shape=jax.ShapeDtypeStruct((NG, N_PAD, ROWW), jnp.float32),
    )(agg, z, deg, bias, WdT)


def _k4_body(agg_ref, z_ref, deg_ref, b2_ref,
             wih0_ref, whh0_ref, bih0_ref, bhh0_ref,
             wih1_ref, whh1_ref, bih1_ref, bhh1_ref,
             w1d_ref, b1d_ref, w2d_ref, b2d_ref, out_ref):
    f32 = jnp.float32
    dinv = lax.rsqrt(deg_ref[:, 0:1] + 1.0)
    h3 = [jnp.maximum((agg_ref[g] + z_ref[g]) * dinv + b2_ref[0], 0.0)
          for g in range(NG)]
    h1 = jnp.zeros((BN, GH), f32)
    h2 = jnp.zeros((BN, GH), f32)
    for t in range(T_WIN):
        g, hf = divmod(t, 2)
        xt = h3[g][:, hf * HID:(hf + 1) * HID]
        gi = jnp.dot(xt, wih0_ref[...], preferred_element_type=f32) + bih0_ref[0]
        gh = jnp.dot(h1, whh0_ref[...], preferred_element_type=f32) + bhh0_ref[0]
        r = jax.nn.sigmoid(gi[:, 0:GH] + gh[:, 0:GH])
        zg = jax.nn.sigmoid(gi[:, GH:2 * GH] + gh[:, GH:2 * GH])
        n = jnp.tanh(gi[:, 2 * GH:3 * GH] + r * gh[:, 2 * GH:3 * GH])
        h1 = (1.0 - zg) * n + zg * h1
        gi2 = jnp.dot(h1, wih1_ref[...], preferred_element_type=f32) + bih1_ref[0]
        gh2 = jnp.dot(h2, whh1_ref[...], preferred_element_type=f32) + bhh1_ref[0]
        r2 = jax.nn.sigmoid(gi2[:, 0:GH] + gh2[:, 0:GH])
        zg2 = jax.nn.sigmoid(gi2[:, GH:2 * GH] + gh2[:, GH:2 * GH])
        n2 = jnp.tanh(gi2[:, 2 * GH:3 * GH] + r2 * gh2[:, 2 * GH:3 * GH])
        h2 = (1.0 - zg2) * n2 + zg2 * h2
        hd = jnp.maximum(jnp.dot(h2, w1d_ref[...], preferred_element_type=f32)
                         + b1d_ref[0], 0.0)
        ot = jnp.dot(hd, w2d_ref[...], preferred_element_type=f32) + b2d_ref[0]
        out_ref[0, :, t, :] = ot


def _k4_call(agg, z, deg, bias2, gw):
    full = lambda *shape: None
    return pl.pallas_call(
        _k4_body,
        grid=(NB,),
        in_specs=[
            pl.BlockSpec((NG, BN, ROWW), lambda nb: (0, nb, 0)),
            pl.BlockSpec((NG, BN, ROWW), lambda nb: (0, nb, 0)),
            pl.BlockSpec((BN, 128), lambda nb: (nb, 0)),
            pl.BlockSpec((1, ROWW), lambda nb: (0, 0)),
            pl.BlockSpec((HID, 3 * GH), lambda nb: (0, 0)),
            pl.BlockSpec((GH, 3 * GH), lambda nb: (0, 0)),
            pl.BlockSpec((1, 3 * GH), lambda nb: (0, 0)),
            pl.BlockSpec((1, 3 * GH), lambda nb: (0, 0)),
            pl.BlockSpec((GH, 3 * GH), lambda nb: (0, 0)),
            pl.BlockSpec((GH, 3 * GH), lambda nb: (0, 0)),
            pl.BlockSpec((1, 3 * GH), lambda nb: (0, 0)),
            pl.BlockSpec((1, 3 * GH), lambda nb: (0, 0)),
            pl.BlockSpec((GH, 96), lambda nb: (0, 0)),
            pl.BlockSpec((1, 96), lambda nb: (0, 0)),
            pl.BlockSpec((96, 12), lambda nb: (0, 0)),
            pl.BlockSpec((1, 12), lambda nb: (0, 0)),
        ],
        out_specs=pl.BlockSpec((1, BN, T_WIN, 12), lambda nb: (0, nb, 0, 0)),
        out_shape=jax.ShapeDtypeStruct((1, N_NODES, T_WIN, 12), jnp.float32),
    )(agg, z, deg, bias2, *gw)


def kernel(x, edge_index, proj_W, proj_b, gcn_W, gcn_b,
           gW_ih0, gW_hh0, gb_ih0, gb_hh0, gW_ih1, gW_hh1, gb_ih1, gb_hh1,
           dec_W1, dec_b1, dec_W2, dec_b2):
    f32 = jnp.float32
    x = x.astype(f32)
    src = edge_index[0].astype(jnp.int32)
    dst = edge_index[1].astype(jnp.int32)
    E = src.shape[0]
    pad = E_PAD - E
    src_p = jnp.concatenate([src, jnp.zeros((pad,), jnp.int32)])
    dst_p = jnp.concatenate([dst, jnp.full((pad,), N_NODES, jnp.int32)])
    dst_idx = dst_p.reshape(N_TILES, NPASS, NCHH, 128)
    goff = (jnp.arange(NG, dtype=jnp.int32) * N_PAD).reshape(GSTEPS, 2, 1, 1, 1, 1)
    src_idx = src_p.reshape(1, 1, N_TILES, NPASS, NCHH, 128) + goff
    zeros128 = jnp.zeros((128, 128), f32)
    ones128 = jnp.ones((128, 128), f32)

    deg = _deg_call(dst_idx, ones128, zeros128)[0]

    Wc = proj_W.T @ gcn_W[0].T
    bc = (proj_b @ gcn_W[0].T).reshape(1, HID)
    Z = _k1_call(x, Wc, bc, deg)
    for l in (0, 1):
        AGG = _agg_call(Z.reshape(NG * N_PAD, ROWW), src_idx, dst_idx, zeros128)
        WdT = jnp.kron(jnp.eye(2, dtype=f32), gcn_W[l + 1].T)
        bias = jnp.tile(gcn_b[l], 2).reshape(1, ROWW)
        Z = _k3_call(AGG, Z, deg, bias, WdT)
    AGG = _agg_call(Z.reshape(NG * N_PAD, ROWW), src_idx, dst_idx, zeros128)
    bias2 = jnp.tile(gcn_b[2], 2).reshape(1, ROWW)

    import jax.scipy.linalg as jsl
    segs = [(0, 1), (1, 2), (2, 3), (3, 6), (6, 9), (9, 12)]
    gw = [
        gW_ih0.T, gW_hh0.T, gb_ih0.reshape(1, -1), gb_hh0.reshape(1, -1),
        gW_ih1.T, gW_hh1.T, gb_ih1.reshape(1, -1), gb_hh1.reshape(1, -1),
        dec_W1.reshape(6 * (GH // 2), GH).T, dec_b1.reshape(1, -1),
        jsl.block_diag(*[dec_W2[s0:s1].T for (s0, s1) in segs]),
        dec_b2.reshape(1, 12),
    ]
    return _k4_call(AGG, Z, deg, bias2, gw)


# speedup vs baseline: 1.2550x; 1.0373x over previous
import functools

import jax
import jax.numpy as jnp
from jax import lax
from jax.experimental import pallas as pl
from jax.experimental.pallas import tpu as pltpu
from jax.experimental.pallas import tpu_sc as plsc

N_NODES = 10000
N_TILES = 16
ROWS_PER_TILE = 640
N_PAD = N_TILES * ROWS_PER_TILE
T_WIN = 24
FEAT = 128
HID = 64
GH = 32
NG = T_WIN // 2
GSTEPS = 3
NGH = NG // 2
ROWW = 2 * HID
NPASS = 2
NCHH = 79
NCH = NPASS * NCHH
E_PAD = N_TILES * NCH * 128
BN = 400
NB = N_NODES // BN


def _sc_mesh():
    return plsc.VectorSubcoreMesh(core_axis_name="c", subcore_axis_name="s")


def _deg_body(dst_hbm, ones_hbm, zero_hbm, out_hbm,
              dst_v, ones_v, zero_v, deg_sh, sem):
    c = lax.axis_index("c")
    s = lax.axis_index("s")
    pltpu.sync_copy(ones_hbm, ones_v)
    pltpu.sync_copy(zero_hbm, zero_v)
    row0 = s * ROWS_PER_TILE
    for k in range(ROWS_PER_TILE // 128):
        pltpu.sync_copy(zero_v, deg_sh.at[pl.ds(row0 + k * 128, 128)])
    plsc.subcore_barrier()

    def chunk(j, carry):
        pltpu.async_copy(ones_v, deg_sh.at[dst_v.at[j]], sem, add=True).wait()
        return carry

    for p in range(NPASS):
        pltpu.sync_copy(dst_hbm.at[s, p], dst_v)
        lax.fori_loop(0, NCHH, chunk, 0)
    plsc.subcore_barrier()

    for k in range(ROWS_PER_TILE // 128):
        r = row0 + k * 128
        pltpu.sync_copy(deg_sh.at[pl.ds(r, 128)], out_hbm.at[c, pl.ds(r, 128)])


def _deg_call(dst_idx, ones128, zeros128):
    return pl.kernel(
        _deg_body,
        out_type=jax.ShapeDtypeStruct((2, N_PAD, 128), jnp.float32),
        mesh=_sc_mesh(),
        scratch_types=[
            pltpu.VMEM((NCHH, 128), jnp.int32),
            pltpu.VMEM((128, 128), jnp.float32),
            pltpu.VMEM((128, 128), jnp.float32),
            pltpu.VMEM_SHARED((N_PAD, 128), jnp.float32),
            pltpu.SemaphoreType.DMA,
        ],
    )(dst_idx, ones128, zeros128)


def _agg_body(z_hbm, src_hbm, dst_hbm, zero_hbm, out_hbm,
              src_v, dst_v, rows_v, agg_sh, gsem, ssem):
    c = lax.axis_index("c")
    s = lax.axis_index("s")
    row0 = s * ROWS_PER_TILE
    for gstep in range(GSTEPS):
        g = gstep * 2 + c
        for k in range(ROWS_PER_TILE // 128):
            pltpu.sync_copy(zero_hbm, agg_sh.at[pl.ds(row0 + k * 128, 128)])
        plsc.subcore_barrier()

        def chunk(j, carry):
            pltpu.async_copy(z_hbm.at[src_v.at[j]], rows_v, gsem).wait()
            pltpu.async_copy(rows_v, agg_sh.at[dst_v.at[j]], ssem,
                             add=True).wait()
            return carry

        for p in range(NPASS):
            pltpu.sync_copy(src_hbm.at[gstep, c, s, p], src_v)
            pltpu.sync_copy(dst_hbm.at[s, p], dst_v)
            lax.fori_loop(0, NCHH, chunk, 0)
        plsc.subcore_barrier()
        for k in range(ROWS_PER_TILE // 128):
            r = row0 + k * 128
            pltpu.sync_copy(agg_sh.at[pl.ds(r, 128)], out_hbm.at[g, pl.ds(r, 128)])


@functools.lru_cache(maxsize=None)
def _agg_kernel():
    return pl.kernel(
        _agg_body,
        out_type=jax.ShapeDtypeStruct((NGH, N_PAD, ROWW), jnp.float32),
        mesh=_sc_mesh(),
        scratch_types=[
            pltpu.VMEM((NCHH, 128), jnp.int32),
            pltpu.VMEM((NCHH, 128), jnp.int32),
            pltpu.VMEM((128, ROWW), jnp.float32),
            pltpu.VMEM_SHARED((N_PAD, ROWW), jnp.float32),
            pltpu.SemaphoreType.DMA,
            pltpu.SemaphoreType.DMA,
        ],
    )


def _agg_call(z_flat, src_idx, dst_idx, zeros128):
    return _agg_kernel()(z_flat, src_idx, dst_idx, zeros128)


def _k1_body(x_ref, wc_ref, bc_ref, deg_ref, out_a_ref, out_b_ref):
    dinv = lax.rsqrt(deg_ref[:, 0:1] + 1.0)
    xb = x_ref[0]
    for t in range(T_WIN):
        z = jnp.dot(xb[:, t, :], wc_ref[...],
                    preferred_element_type=jnp.float32) + bc_ref[0]
        z = z * dinv
        g, h = divmod(t, 2)
        o = out_a_ref if g < NGH else out_b_ref
        o[g % NGH, :, h * HID:(h + 1) * HID] = z


def _k1_call(x, Wc, bc, deg):
    return pl.pallas_call(
        _k1_body,
        grid=(NB,),
        in_specs=[
            pl.BlockSpec((1, BN, T_WIN, FEAT), lambda nb: (0, nb, 0, 0)),
            pl.BlockSpec((FEAT, HID), lambda nb: (0, 0)),
            pl.BlockSpec((1, HID), lambda nb: (0, 0)),
            pl.BlockSpec((BN, 128), lambda nb: (nb, 0)),
        ],
        out_specs=[pl.BlockSpec((NGH, BN, ROWW), lambda nb: (0, nb, 0)),
                   pl.BlockSpec((NGH, BN, ROWW), lambda nb: (0, nb, 0))],
        out_shape=[jax.ShapeDtypeStruct((NGH, N_PAD, ROWW), jnp.float32),
                   jax.ShapeDtypeStruct((NGH, N_PAD, ROWW), jnp.float32)],
    )(x, Wc, bc, deg)


def _k3_body(agg_ref, z_ref, deg_ref, b_ref, wd_ref, out_ref):
    dinv = lax.rsqrt(deg_ref[:, 0:1] + 1.0)
    h = jnp.maximum((agg_ref[0] + z_ref[0]) * dinv + b_ref[0], 0.0)
    out_ref[0] = jnp.dot(h, wd_ref[...],
                         preferred_element_type=jnp.float32) * dinv


def _k3_call(agg, z, deg, bias, WdT):
    return pl.pallas_call(
        _k3_body,
        grid=(NGH, NB),
        in_specs=[
            pl.BlockSpec((1, BN, ROWW), lambda g, nb: (g, nb, 0)),
            pl.BlockSpec((1, BN, ROWW), lambda g, nb: (g, nb, 0)),
            pl.BlockSpec((BN, 128), lambda g, nb: (nb, 0)),
            pl.BlockSpec((1, ROWW), lambda g, nb: (0, 0)),
            pl.BlockSpec((ROWW, ROWW), lambda g, nb: (0, 0)),
        ],
        out_specs=pl.BlockSpec((1, BN, ROWW), lambda g, nb: (g, nb, 0)),
        out_shape=jax.ShapeDtypeStruct((NGH, N_PAD, ROWW), jnp.float32),
    )(agg, z, deg, bias, WdT)


def _k4_body(agg_a_ref, z_a_ref, agg_b_ref, z_b_ref, deg_ref, b2_ref,
             wih0_ref, whh0_ref, bih0_ref, bhh0_ref,
             wih1_ref, whh1_ref, bih1_ref, bhh1_ref,
             w1d_ref, b1d_ref, w2d_ref, b2d_ref, out_ref):
    f32 = jnp.float32
    dinv = lax.rsqrt(deg_ref[:, 0:1] + 1.0)
    h3 = [jnp.maximum((agg_a_ref[g] + z_a_ref[g]) * dinv + b2_ref[0], 0.0)
          for g in range(NGH)]
    h3 += [jnp.maximum((agg_b_ref[g] + z_b_ref[g]) * dinv + b2_ref[0], 0.0)
           for g in range(NGH)]
    h1 = jnp.zeros((BN, GH), f32)
    h2 = jnp.zeros((BN, GH), f32)
    for t in range(T_WIN):
        g, hf = divmod(t, 2)
        xt = h3[g][:, hf * HID:(hf + 1) * HID]
        gi = jnp.dot(xt, wih0_ref[...], preferred_element_type=f32) + bih0_ref[0]
        gh = jnp.dot(h1, whh0_ref[...], preferred_element_type=f32) + bhh0_ref[0]
        r = jax.nn.sigmoid(gi[:, 0:GH] + gh[:, 0:GH])
        zg = jax.nn.sigmoid(gi[:, GH:2 * GH] + gh[:, GH:2 * GH])
        n = jnp.tanh(gi[:, 2 * GH:3 * GH] + r * gh[:, 2 * GH:3 * GH])
        h1 = (1.0 - zg) * n + zg * h1
        gi2 = jnp.dot(h1, wih1_ref[...], preferred_element_type=f32) + bih1_ref[0]
        gh2 = jnp.dot(h2, whh1_ref[...], preferred_element_type=f32) + bhh1_ref[0]
        r2 = jax.nn.sigmoid(gi2[:, 0:GH] + gh2[:, 0:GH])
        zg2 = jax.nn.sigmoid(gi2[:, GH:2 * GH] + gh2[:, GH:2 * GH])
        n2 = jnp.tanh(gi2[:, 2 * GH:3 * GH] + r2 * gh2[:, 2 * GH:3 * GH])
        h2 = (1.0 - zg2) * n2 + zg2 * h2
        hd = jnp.maximum(jnp.dot(h2, w1d_ref[...], preferred_element_type=f32)
                         + b1d_ref[0], 0.0)
        ot = jnp.dot(hd, w2d_ref[...], preferred_element_type=f32) + b2d_ref[0]
        out_ref[0, :, t, :] = ot


def _k4_call(agg_a, z_a, agg_b, z_b, deg, bias2, gw):
    return pl.pallas_call(
        _k4_body,
        grid=(NB,),
        in_specs=[
            pl.BlockSpec((NGH, BN, ROWW), lambda nb: (0, nb, 0)),
            pl.BlockSpec((NGH, BN, ROWW), lambda nb: (0, nb, 0)),
            pl.BlockSpec((NGH, BN, ROWW), lambda nb: (0, nb, 0)),
            pl.BlockSpec((NGH, BN, ROWW), lambda nb: (0, nb, 0)),
            pl.BlockSpec((BN, 128), lambda nb: (nb, 0)),
            pl.BlockSpec((1, ROWW), lambda nb: (0, 0)),
            pl.BlockSpec((HID, 3 * GH), lambda nb: (0, 0)),
            pl.BlockSpec((GH, 3 * GH), lambda nb: (0, 0)),
            pl.BlockSpec((1, 3 * GH), lambda nb: (0, 0)),
            pl.BlockSpec((1, 3 * GH), lambda nb: (0, 0)),
            pl.BlockSpec((GH, 3 * GH), lambda nb: (0, 0)),
            pl.BlockSpec((GH, 3 * GH), lambda nb: (0, 0)),
            pl.BlockSpec((1, 3 * GH), lambda nb: (0, 0)),
            pl.BlockSpec((1, 3 * GH), lambda nb: (0, 0)),
            pl.BlockSpec((GH, 96), lambda nb: (0, 0)),
            pl.BlockSpec((1, 96), lambda nb: (0, 0)),
            pl.BlockSpec((96, 12), lambda nb: (0, 0)),
            pl.BlockSpec((1, 12), lambda nb: (0, 0)),
        ],
        out_specs=pl.BlockSpec((1, BN, T_WIN, 12), lambda nb: (0, nb, 0, 0)),
        out_shape=jax.ShapeDtypeStruct((1, N_NODES, T_WIN, 12), jnp.float32),
    )(agg_a, z_a, agg_b, z_b, deg, bias2, *gw)


def kernel(x, edge_index, proj_W, proj_b, gcn_W, gcn_b,
           gW_ih0, gW_hh0, gb_ih0, gb_hh0, gW_ih1, gW_hh1, gb_ih1, gb_hh1,
           dec_W1, dec_b1, dec_W2, dec_b2):
    f32 = jnp.float32
    x = x.astype(f32)
    src = edge_index[0].astype(jnp.int32)
    dst = edge_index[1].astype(jnp.int32)
    E = src.shape[0]
    pad = E_PAD - E
    src_p = jnp.concatenate([src, jnp.zeros((pad,), jnp.int32)])
    dst_p = jnp.concatenate([dst, jnp.full((pad,), N_NODES, jnp.int32)])
    dst_idx = dst_p.reshape(N_TILES, NPASS, NCHH, 128)
    goff = (jnp.arange(NGH, dtype=jnp.int32) * N_PAD).reshape(GSTEPS, 2, 1, 1, 1, 1)
    src_idx = src_p.reshape(1, 1, N_TILES, NPASS, NCHH, 128) + goff
    zeros128 = jnp.zeros((128, 128), f32)
    ones128 = jnp.ones((128, 128), f32)

    deg = _deg_call(dst_idx, ones128, zeros128)[0]

    Wc = proj_W.T @ gcn_W[0].T
    bc = (proj_b @ gcn_W[0].T).reshape(1, HID)
    ZA, ZB = _k1_call(x, Wc, bc, deg)
    for l in (0, 1):
        WdT = jnp.kron(jnp.eye(2, dtype=f32), gcn_W[l + 1].T)
        bias = jnp.tile(gcn_b[l], 2).reshape(1, ROWW)
        AA = _agg_call(ZA.reshape(NGH * N_PAD, ROWW), src_idx, dst_idx, zeros128)
        AB = _agg_call(ZB.reshape(NGH * N_PAD, ROWW), src_idx, dst_idx, zeros128)
        ZA = _k3_call(AA, ZA, deg, bias, WdT)
        ZB = _k3_call(AB, ZB, deg, bias, WdT)
    AA = _agg_call(ZA.reshape(NGH * N_PAD, ROWW), src_idx, dst_idx, zeros128)
    AB = _agg_call(ZB.reshape(NGH * N_PAD, ROWW), src_idx, dst_idx, zeros128)
    bias2 = jnp.tile(gcn_b[2], 2).reshape(1, ROWW)

    import jax.scipy.linalg as jsl
    segs = [(0, 1), (1, 2), (2, 3), (3, 6), (6, 9), (9, 12)]
    gw = [
        gW_ih0.T, gW_hh0.T, gb_ih0.reshape(1, -1), gb_hh0.reshape(1, -1),
        gW_ih1.T, gW_hh1.T, gb_ih1.reshape(1, -1), gb_hh1.reshape(1, -1),
        dec_W1.reshape(6 * (GH // 2), GH).T, dec_b1.reshape(1, -1),
        jsl.block_diag(*[dec_W2[s0:s1].T for (s0, s1) in segs]),
        dec_b2.reshape(1, 12),
    ]
    return _k4_call(AA, ZA, AB, ZB, deg, bias2, gw)
